# Initial kernel scaffold; baseline (speedup 1.0000x reference)
import jax
import jax.numpy as jnp
from jax.experimental import pallas as pl


def kernel(H, edge_index, W, b):
    raise NotImplementedError("write your pallas kernel here")


# speedup vs baseline: 18.0195x; 18.0195x over previous
import functools

import jax
import jax.numpy as jnp
from jax import lax
from jax.experimental import pallas as pl
from jax.experimental.pallas import tpu as pltpu
from jax.experimental.pallas import tpu_sc as plsc

N = 10000
E = 320000
D = 128
NC = 2
NS = 16
NW = NC * NS
EW = E // NW
CH = 80
NWIN = EW // CH
NP = 10240
RPS = NP // NS
BR = 1000


def _mesh():
    return plsc.VectorSubcoreMesh(core_axis_name="c", subcore_axis_name="s")


def _sc_degree(row):
    @functools.partial(
        pl.kernel,
        mesh=_mesh(),
        out_type=jax.ShapeDtypeStruct((NC * NP,), jnp.float32),
        scratch_types=[
            pltpu.VMEM((CH,), jnp.int32),
            pltpu.VMEM((CH,), jnp.float32),
            pltpu.VMEM((RPS,), jnp.float32),
            pltpu.VMEM_SHARED((NP,), jnp.float32),
        ],
    )
    def k(row_hbm, out_hbm, idx_v, ones_v, tb_v, hist_sh):
        c = lax.axis_index("c")
        s = lax.axis_index("s")
        wid = s * NC + c

        def fill_ones(i, carry):
            ones_v[pl.ds(i * 16, 16)] = jnp.ones((16,), jnp.float32)
            return carry

        lax.fori_loop(0, CH // 16, fill_ones, 0)

        def zero_tb(i, carry):
            tb_v[pl.ds(i * 16, 16)] = jnp.zeros((16,), jnp.float32)
            return carry

        lax.fori_loop(0, RPS // 16, zero_tb, 0)
        pltpu.sync_copy(tb_v, hist_sh.at[pl.ds(s * RPS, RPS)])
        plsc.subcore_barrier()

        base = wid * EW

        def body(w, carry):
            pltpu.sync_copy(row_hbm.at[pl.ds(base + w * CH, CH)], idx_v)
            pltpu.sync_copy(ones_v, hist_sh.at[idx_v], add=True)
            return carry

        lax.fori_loop(0, NWIN, body, 0)
        plsc.subcore_barrier()

        pltpu.sync_copy(hist_sh.at[pl.ds(s * RPS, RPS)], tb_v)
        pltpu.sync_copy(tb_v, out_hbm.at[pl.ds(c * NP + s * RPS, RPS)])

    return k(row)


def _sc_scatter(G, row, col):
    @functools.partial(
        pl.kernel,
        mesh=_mesh(),
        out_type=jax.ShapeDtypeStruct((NC * NP, D), jnp.float32),
        scratch_types=[
            pltpu.VMEM((CH,), jnp.int32),
            pltpu.VMEM((CH,), jnp.int32),
            pltpu.VMEM((CH, D), jnp.float32),
            pltpu.VMEM_SHARED((NP, D), jnp.float32),
            pltpu.SemaphoreType.DMA,
        ],
    )
    def k(g_hbm, row_hbm, col_hbm, out_hbm, cidx, ridx, rbuf, acc, sem):
        c = lax.axis_index("c")
        s = lax.axis_index("s")
        wid = s * NC + c

        def zrow(r, carry):
            def zlane(j, carry2):
                rbuf[r, pl.ds(j * 16, 16)] = jnp.zeros((16,), jnp.float32)
                return carry2

            lax.fori_loop(0, D // 16, zlane, 0)
            return carry

        lax.fori_loop(0, CH, zrow, 0)

        def zcopy(t, carry):
            pltpu.sync_copy(rbuf, acc.at[pl.ds(s * RPS + t * CH, CH)])
            return carry

        lax.fori_loop(0, RPS // CH, zcopy, 0)
        plsc.subcore_barrier()

        base = wid * EW

        def body(w, carry):
            pltpu.sync_copy(col_hbm.at[pl.ds(base + w * CH, CH)], cidx)
            pltpu.async_copy(g_hbm.at[cidx], rbuf, sem).wait()
            pltpu.sync_copy(row_hbm.at[pl.ds(base + w * CH, CH)], ridx)
            pltpu.sync_copy(rbuf, acc.at[ridx], add=True)
            return carry

        lax.fori_loop(0, NWIN, body, 0)
        plsc.subcore_barrier()

        def epil(t, carry):
            pltpu.sync_copy(acc.at[pl.ds(s * RPS + t * CH, CH)], rbuf)
            pltpu.sync_copy(rbuf, out_hbm.at[pl.ds(c * NP + s * RPS + t * CH, CH)])
            return carry

        lax.fori_loop(0, RPS // CH, epil, 0)

    return k(G, row, col)


def _tc_transform(H, Wm, b2, histT):
    def body(h_ref, w_ref, b_ref, ht_ref, g_ref):
        hl = lax.dot_general(
            h_ref[...], w_ref[...], (((1,), (1,)), ((), ())),
            preferred_element_type=jnp.float32,
        ) + b_ref[...]
        deg = ht_ref[:, 0:1] + ht_ref[:, 1:2] + 1.0
        g_ref[...] = hl * lax.rsqrt(deg)

    return pl.pallas_call(
        body,
        grid=(N // BR,),
        in_specs=[
            pl.BlockSpec((BR, D), lambda k: (k, 0)),
            pl.BlockSpec((D, D), lambda k: (0, 0)),
            pl.BlockSpec((1, D), lambda k: (0, 0)),
            pl.BlockSpec((BR, NC), lambda k: (k, 0)),
        ],
        out_specs=pl.BlockSpec((BR, D), lambda k: (k, 0)),
        out_shape=jax.ShapeDtypeStruct((N, D), jnp.float32),
    )(H, Wm, b2, histT)


def _tc_finish(P, G, histT):
    def body(p_ref, g_ref, ht_ref, o_ref):
        accsum = p_ref[0] + p_ref[1] + g_ref[...]
        deg = ht_ref[:, 0:1] + ht_ref[:, 1:2] + 1.0
        o_ref[...] = jnp.maximum(accsum * lax.rsqrt(deg), 0.0)

    return pl.pallas_call(
        body,
        grid=(N // BR,),
        in_specs=[
            pl.BlockSpec((NC, BR, D), lambda k: (0, k, 0)),
            pl.BlockSpec((BR, D), lambda k: (k, 0)),
            pl.BlockSpec((BR, NC), lambda k: (k, 0)),
        ],
        out_specs=pl.BlockSpec((BR, D), lambda k: (k, 0)),
        out_shape=jax.ShapeDtypeStruct((N, D), jnp.float32),
    )(P, G, histT)


def kernel(H, edge_index, W, b):
    ei = edge_index.astype(jnp.int32)
    row = ei[0]
    col = ei[1]
    hist = _sc_degree(row)
    histT = hist.reshape(NC, NP).T
    G = _tc_transform(H, W, b.reshape(1, D), histT)
    P = _sc_scatter(G, row, col).reshape(NC, NP, D)
    return _tc_finish(P, G, histT)


# speedup vs baseline: 34.3394x; 1.9057x over previous
import functools

import jax
import jax.numpy as jnp
from jax import lax
from jax.experimental import pallas as pl
from jax.experimental.pallas import tpu as pltpu
from jax.experimental.pallas import tpu_sc as plsc

N = 10000
E = 320000
D = 128
NC = 2
NS = 16
NW = NC * NS
EW = E // NW
CH = 80
NWIN = EW // CH
NP = 10240
RPS = NP // NS
BR = 1000


def _mesh():
    return plsc.VectorSubcoreMesh(core_axis_name="c", subcore_axis_name="s")


def _sc_degree(row):
    @functools.partial(
        pl.kernel,
        mesh=_mesh(),
        out_type=jax.ShapeDtypeStruct((NC * NP,), jnp.float32),
        scratch_types=[
            pltpu.VMEM((EW,), jnp.int32),
            pltpu.VMEM((CH,), jnp.float32),
            pltpu.VMEM((RPS,), jnp.float32),
            pltpu.VMEM_SHARED((NP,), jnp.float32),
        ],
    )
    def k(row_hbm, out_hbm, idx_v, ones_v, tb_v, hist_sh):
        c = lax.axis_index("c")
        s = lax.axis_index("s")
        wid = s * NC + c

        def fill_ones(i, carry):
            ones_v[pl.ds(i * 16, 16)] = jnp.ones((16,), jnp.float32)
            return carry

        lax.fori_loop(0, CH // 16, fill_ones, 0)

        def zero_tb(i, carry):
            tb_v[pl.ds(i * 16, 16)] = jnp.zeros((16,), jnp.float32)
            return carry

        lax.fori_loop(0, RPS // 16, zero_tb, 0)
        pltpu.sync_copy(tb_v, hist_sh.at[pl.ds(s * RPS, RPS)])
        pltpu.sync_copy(row_hbm.at[pl.ds(wid * EW, EW)], idx_v)
        plsc.subcore_barrier()

        def body(w, carry):
            pltpu.sync_copy(ones_v, hist_sh.at[idx_v.at[pl.ds(w * CH, CH)]], add=True)
            return carry

        lax.fori_loop(0, NWIN, body, 0)
        plsc.subcore_barrier()

        pltpu.sync_copy(hist_sh.at[pl.ds(s * RPS, RPS)], tb_v)
        pltpu.sync_copy(tb_v, out_hbm.at[pl.ds(c * NP + s * RPS, RPS)])

    return k(row)


def _sc_scatter(G, row, col):
    @functools.partial(
        pl.kernel,
        mesh=_mesh(),
        out_type=jax.ShapeDtypeStruct((NC * NP, D), jnp.float32),
        scratch_types=[
            pltpu.VMEM((EW,), jnp.int32),
            pltpu.VMEM((EW,), jnp.int32),
            pltpu.VMEM((2, CH, D), jnp.float32),
            pltpu.VMEM_SHARED((NP, D), jnp.float32),
            pltpu.SemaphoreType.DMA,
        ],
    )
    def k(g_hbm, row_hbm, col_hbm, out_hbm, cidx, ridx, rbuf, acc, gsem):
        c = lax.axis_index("c")
        s = lax.axis_index("s")
        wid = s * NC + c

        def zrow(r, carry):
            def zlane(j, carry2):
                rbuf[0, r, pl.ds(j * 16, 16)] = jnp.zeros((16,), jnp.float32)
                return carry2

            lax.fori_loop(0, D // 16, zlane, 0)
            return carry

        lax.fori_loop(0, CH, zrow, 0)

        def zcopy(t, carry):
            pltpu.sync_copy(rbuf.at[0], acc.at[pl.ds(s * RPS + t * CH, CH)])
            return carry

        lax.fori_loop(0, RPS // CH, zcopy, 0)

        pltpu.sync_copy(col_hbm.at[pl.ds(wid * EW, EW)], cidx)
        pltpu.sync_copy(row_hbm.at[pl.ds(wid * EW, EW)], ridx)
        plsc.subcore_barrier()

        pltpu.async_copy(g_hbm.at[cidx.at[pl.ds(0, CH)]], rbuf.at[0], gsem)

        def body(w, carry):
            b = lax.rem(w, 2)
            pltpu.make_async_copy(g_hbm.at[cidx.at[pl.ds(w * CH, CH)]], rbuf.at[b], gsem).wait()

            @pl.when(w + 1 < NWIN)
            def _():
                pltpu.async_copy(g_hbm.at[cidx.at[pl.ds((w + 1) * CH, CH)]], rbuf.at[1 - b], gsem)

            pltpu.sync_copy(rbuf.at[b], acc.at[ridx.at[pl.ds(w * CH, CH)]], add=True)
            return carry

        lax.fori_loop(0, NWIN, body, 0)
        plsc.subcore_barrier()

        def epil(t, carry):
            pltpu.sync_copy(acc.at[pl.ds(s * RPS + t * CH, CH)], rbuf.at[0])
            pltpu.sync_copy(rbuf.at[0], out_hbm.at[pl.ds(c * NP + s * RPS + t * CH, CH)])
            return carry

        lax.fori_loop(0, RPS // CH, epil, 0)

    return k(G, row, col)


def _tc_transform(H, Wm, b2, histT):
    def body(h_ref, w_ref, b_ref, ht_ref, g_ref):
        hl = lax.dot_general(
            h_ref[...], w_ref[...], (((1,), (1,)), ((), ())),
            preferred_element_type=jnp.float32,
        ) + b_ref[...]
        deg = ht_ref[:, 0:1] + ht_ref[:, 1:2] + 1.0
        g_ref[...] = hl * lax.rsqrt(deg)

    return pl.pallas_call(
        body,
        grid=(N // BR,),
        in_specs=[
            pl.BlockSpec((BR, D), lambda k: (k, 0)),
            pl.BlockSpec((D, D), lambda k: (0, 0)),
            pl.BlockSpec((1, D), lambda k: (0, 0)),
            pl.BlockSpec((BR, NC), lambda k: (k, 0)),
        ],
        out_specs=pl.BlockSpec((BR, D), lambda k: (k, 0)),
        out_shape=jax.ShapeDtypeStruct((N, D), jnp.float32),
    )(H, Wm, b2, histT)


def _tc_finish(P, G, histT):
    def body(p_ref, g_ref, ht_ref, o_ref):
        accsum = p_ref[0] + p_ref[1] + g_ref[...]
        deg = ht_ref[:, 0:1] + ht_ref[:, 1:2] + 1.0
        o_ref[...] = jnp.maximum(accsum * lax.rsqrt(deg), 0.0)

    return pl.pallas_call(
        body,
        grid=(N // BR,),
        in_specs=[
            pl.BlockSpec((NC, BR, D), lambda k: (0, k, 0)),
            pl.BlockSpec((BR, D), lambda k: (k, 0)),
            pl.BlockSpec((BR, NC), lambda k: (k, 0)),
        ],
        out_specs=pl.BlockSpec((BR, D), lambda k: (k, 0)),
        out_shape=jax.ShapeDtypeStruct((N, D), jnp.float32),
    )(P, G, histT)


def kernel(H, edge_index, W, b):
    ei = edge_index.astype(jnp.int32)
    row = ei[0]
    col = ei[1]
    hist = _sc_degree(row)
    histT = hist.reshape(NC, NP).T
    G = _tc_transform(H, W, b.reshape(1, D), histT)
    P = _sc_scatter(G, row, col).reshape(NC, NP, D)
    return _tc_finish(P, G, histT)


# speedup vs baseline: 39.3131x; 1.1448x over previous
import functools

import jax
import jax.numpy as jnp
from jax import lax
from jax.experimental import pallas as pl
from jax.experimental.pallas import tpu as pltpu
from jax.experimental.pallas import tpu_sc as plsc

N = 10000
E = 320000
D = 128
NC = 2
NS = 16
NW = NC * NS
EW = E // NW
CH = 80
NWIN = EW // CH
CHC = 128
NFULL = EW // CHC
TAIL = EW - NFULL * CHC
NB = 4
NP = 10240
RPS = NP // NS
BR = 1000


def _mesh():
    return plsc.VectorSubcoreMesh(core_axis_name="c", subcore_axis_name="s")


def _sc_degree(row):
    @functools.partial(
        pl.kernel,
        mesh=_mesh(),
        out_type=jax.ShapeDtypeStruct((NC * NP,), jnp.float32),
        scratch_types=[
            pltpu.VMEM((EW,), jnp.int32),
            pltpu.VMEM((CH,), jnp.float32),
            pltpu.VMEM((RPS,), jnp.float32),
            pltpu.VMEM_SHARED((NP,), jnp.float32),
        ],
    )
    def k(row_hbm, out_hbm, idx_v, ones_v, tb_v, hist_sh):
        c = lax.axis_index("c")
        s = lax.axis_index("s")
        wid = s * NC + c

        def fill_ones(i, carry):
            ones_v[pl.ds(i * 16, 16)] = jnp.ones((16,), jnp.float32)
            return carry

        lax.fori_loop(0, CH // 16, fill_ones, 0)

        def zero_tb(i, carry):
            tb_v[pl.ds(i * 16, 16)] = jnp.zeros((16,), jnp.float32)
            return carry

        lax.fori_loop(0, RPS // 16, zero_tb, 0)
        pltpu.sync_copy(tb_v, hist_sh.at[pl.ds(s * RPS, RPS)])
        pltpu.sync_copy(row_hbm.at[pl.ds(wid * EW, EW)], idx_v)
        plsc.subcore_barrier()

        def body(w, carry):
            pltpu.sync_copy(ones_v, hist_sh.at[idx_v.at[pl.ds(w * CH, CH)]], add=True)
            return carry

        lax.fori_loop(0, NWIN, body, 0)
        plsc.subcore_barrier()

        pltpu.sync_copy(hist_sh.at[pl.ds(s * RPS, RPS)], tb_v)
        pltpu.sync_copy(tb_v, out_hbm.at[pl.ds(c * NP + s * RPS, RPS)])

    return k(row)


def _sc_scatter(G, row, col):
    @functools.partial(
        pl.kernel,
        mesh=_mesh(),
        out_type=jax.ShapeDtypeStruct((NC * NP, D), jnp.float32),
        scratch_types=[
            pltpu.VMEM((EW,), jnp.int32),
            pltpu.VMEM((CHC,), jnp.int32),
            pltpu.VMEM((CHC,), jnp.int32),
            pltpu.VMEM((2, CHC, D), jnp.float32),
            pltpu.VMEM_SHARED((NP, D), jnp.float32),
            pltpu.SemaphoreType.DMA,
            pltpu.SemaphoreType.DMA,
        ],
    )
    def k(g_hbm, row_hbm, col_hbm, out_hbm, cidx, rref0, rref1, rbuf, acc,
          gsem, rsem):
        c = lax.axis_index("c")
        s = lax.axis_index("s")
        wid = s * NC + c
        base = wid * EW
        rrefs = (rref0, rref1)

        def gather_desc(w, b):
            return pltpu.make_async_copy(
                g_hbm.at[cidx.at[pl.ds(w * CHC, CHC)]], rbuf.at[b], gsem)

        def ridx_desc(w, u):
            return pltpu.make_async_copy(
                row_hbm.at[pl.ds(base + w * CHC, CHC)], rrefs[u], rsem)

        def zrow(r, carry):
            def zlane(j, carry2):
                rbuf[0, r, pl.ds(j * 16, 16)] = jnp.zeros((16,), jnp.float32)
                return carry2

            lax.fori_loop(0, D // 16, zlane, 0)
            return carry

        lax.fori_loop(0, CHC, zrow, 0)

        def zcopy(t, carry):
            pltpu.sync_copy(rbuf.at[0], acc.at[pl.ds(s * RPS + t * CHC, CHC)])
            return carry

        lax.fori_loop(0, RPS // CHC, zcopy, 0)

        pltpu.sync_copy(col_hbm.at[pl.ds(base, EW)], cidx)
        plsc.subcore_barrier()

        ridx_desc(0, 0).start()
        pltpu.async_copy(
            g_hbm.at[cidx.at[pl.ds(0, CHC)]], rbuf.at[0], gsem)

        def body(g, carry):
            for u in (0, 1):
                w = g * 2 + u
                gather_desc(w, u).wait()

                @pl.when(w + 1 < NFULL)
                def _():
                    pltpu.async_copy(
                        g_hbm.at[cidx.at[pl.ds((w + 1) * CHC, CHC)]],
                        rbuf.at[1 - u], gsem)
                    ridx_desc(w + 1, 1 - u).start()

                ridx_desc(w, u).wait()
                pltpu.sync_copy(
                    rbuf.at[u], acc.at[rrefs[u]], add=True)
            return carry

        lax.fori_loop(0, NFULL // 2, body, 0)
        if TAIL:
            t0 = NFULL * CHC
            pltpu.async_copy(
                g_hbm.at[cidx.at[pl.ds(t0, TAIL)]],
                rbuf.at[0, pl.ds(0, TAIL)], gsem).wait()
            pltpu.sync_copy(
                row_hbm.at[pl.ds(base + t0, TAIL)], rref0.at[pl.ds(0, TAIL)])
            pltpu.sync_copy(
                rbuf.at[0, pl.ds(0, TAIL)],
                acc.at[rref0.at[pl.ds(0, TAIL)]], add=True)
        plsc.subcore_barrier()

        def epil(t, carry):
            pltpu.sync_copy(acc.at[pl.ds(s * RPS + t * CHC, CHC)], rbuf.at[0])
            pltpu.sync_copy(rbuf.at[0], out_hbm.at[pl.ds(c * NP + s * RPS + t * CHC, CHC)])
            return carry

        lax.fori_loop(0, RPS // CHC, epil, 0)

    return k(G, row, col)


def _tc_transform(H, Wm, b2, histT):
    def body(h_ref, w_ref, b_ref, ht_ref, g_ref):
        hl = lax.dot_general(
            h_ref[...], w_ref[...], (((1,), (1,)), ((), ())),
            preferred_element_type=jnp.float32,
        ) + b_ref[...]
        deg = ht_ref[:, 0:1] + ht_ref[:, 1:2] + 1.0
        g_ref[...] = hl * lax.rsqrt(deg)

    return pl.pallas_call(
        body,
        grid=(N // BR,),
        in_specs=[
            pl.BlockSpec((BR, D), lambda k: (k, 0)),
            pl.BlockSpec((D, D), lambda k: (0, 0)),
            pl.BlockSpec((1, D), lambda k: (0, 0)),
            pl.BlockSpec((BR, NC), lambda k: (k, 0)),
        ],
        out_specs=pl.BlockSpec((BR, D), lambda k: (k, 0)),
        out_shape=jax.ShapeDtypeStruct((N, D), jnp.float32),
    )(H, Wm, b2, histT)


def _tc_finish(P, G, histT):
    def body(p_ref, g_ref, ht_ref, o_ref):
        accsum = p_ref[0] + p_ref[1] + g_ref[...]
        deg = ht_ref[:, 0:1] + ht_ref[:, 1:2] + 1.0
        o_ref[...] = jnp.maximum(accsum * lax.rsqrt(deg), 0.0)

    return pl.pallas_call(
        body,
        grid=(N // BR,),
        in_specs=[
            pl.BlockSpec((NC, BR, D), lambda k: (0, k, 0)),
            pl.BlockSpec((BR, D), lambda k: (k, 0)),
            pl.BlockSpec((BR, NC), lambda k: (k, 0)),
        ],
        out_specs=pl.BlockSpec((BR, D), lambda k: (k, 0)),
        out_shape=jax.ShapeDtypeStruct((N, D), jnp.float32),
    )(P, G, histT)


def kernel(H, edge_index, W, b):
    ei = edge_index.astype(jnp.int32)
    row = ei[0]
    col = ei[1]
    hist = _sc_degree(row)
    histT = hist.reshape(NC, NP).T
    G = _tc_transform(H, W, b.reshape(1, D), histT)
    P = _sc_scatter(G, row, col).reshape(NC, NP, D)
    return _tc_finish(P, G, histT)
